# baseline (speedup 1.0000x reference)
import jax
import jax.numpy as jnp
from jax import lax
from jax.experimental import pallas as pl
from jax.experimental.pallas import tpu as pltpu
from jax.experimental.pallas import tpu_sc as plsc

N_PAD = 10240
NC = 2
NS = 16
NW = NC * NS
CHUNK = 128
ROWS_PER_TILE = N_PAD // NS
ZROWS = 32
G = 40
DEG_W = 128


def _make_sc_agg(nchunk: int, d: int):
    ngroups = nchunk // G
    assert ngroups * G == nchunk and G % 2 == 0 and G >= 4
    mesh = plsc.VectorSubcoreMesh(
        core_axis_name="c", subcore_axis_name="s", num_cores=NC, num_subcores=NS
    )
    out_type = [jax.ShapeDtypeStruct((NC, N_PAD, d), jnp.float32)]
    scratch = [
        pltpu.VMEM((G, CHUNK), jnp.int32),
        pltpu.VMEM((G, CHUNK), jnp.int32),
        pltpu.VMEM((CHUNK, d), jnp.float32),
        pltpu.VMEM((CHUNK, d), jnp.float32),
        pltpu.VMEM((ZROWS, d), jnp.float32),
        pltpu.VMEM_SHARED((N_PAD, d), jnp.float32),
        pltpu.SemaphoreType.DMA,
        pltpu.SemaphoreType.DMA,
        pltpu.SemaphoreType.DMA,
        pltpu.SemaphoreType.DMA,
    ]

    def body(y_hbm, src_hbm, dst_hbm, out_hbm,
             sbuf, dbuf, rows_a, rows_b, zbuf, acc, ga, gb, sa, sb):
        cid = lax.axis_index("c")
        sid = lax.axis_index("s")
        wid = sid * NC + cid

        zvec = jnp.zeros((16,), jnp.float32)

        def zfill(j, carry):
            for k in range(d // 16):
                zbuf[j, pl.ds(k * 16, 16)] = zvec
            return carry

        lax.fori_loop(0, ZROWS, zfill, 0)

        base = sid * ROWS_PER_TILE
        for z in range(ROWS_PER_TILE // ZROWS):
            pltpu.sync_copy(zbuf, acc.at[pl.ds(base + z * ZROWS, ZROWS)])
        plsc.subcore_barrier()

        def g_start(j, buf, sem):
            pltpu.async_copy(y_hbm.at[sbuf.at[j]], buf, sem)

        def g_wait(j, buf, sem):
            pltpu.make_async_copy(y_hbm.at[sbuf.at[j]], buf, sem).wait()

        def s_start(j, buf, sem):
            pltpu.async_copy(buf, acc.at[dbuf.at[j]], sem, add=True)

        def s_wait(j, buf, sem):
            pltpu.make_async_copy(buf, acc.at[dbuf.at[j]], sem).wait()

        for g in range(ngroups):
            pltpu.sync_copy(src_hbm.at[wid, pl.ds(g * G, G)], sbuf)
            pltpu.sync_copy(dst_hbm.at[wid, pl.ds(g * G, G)], dbuf)
            g_start(0, rows_a, ga)
            g_wait(0, rows_a, ga)
            s_start(0, rows_a, sa)
            g_start(1, rows_b, gb)
            g_wait(1, rows_b, gb)
            s_wait(0, rows_a, sa)
            g_start(2, rows_a, ga)
            s_start(1, rows_b, sb)

            def pair(t, carry):
                j0 = 2 * t
                j1 = j0 + 1
                g_wait(j0, rows_a, ga)
                s_wait(j1 - 2, rows_b, sb)
                g_start(j1, rows_b, gb)
                s_start(j0, rows_a, sa)
                g_wait(j1, rows_b, gb)
                s_wait(j0, rows_a, sa)
                g_start(j0 + 2, rows_a, ga)
                s_start(j1, rows_b, sb)
                return carry

            lax.fori_loop(1, G // 2 - 1, pair, 0)

            j0 = G - 2
            j1 = G - 1
            g_wait(j0, rows_a, ga)
            s_wait(j1 - 2, rows_b, sb)
            g_start(j1, rows_b, gb)
            s_start(j0, rows_a, sa)
            g_wait(j1, rows_b, gb)
            s_wait(j0, rows_a, sa)
            s_start(j1, rows_b, sb)
            s_wait(j1, rows_b, sb)

        plsc.subcore_barrier()

        pltpu.sync_copy(
            acc.at[pl.ds(base, ROWS_PER_TILE)],
            out_hbm.at[cid, pl.ds(base, ROWS_PER_TILE)],
        )

    return pl.kernel(body, out_type=out_type, mesh=mesh, scratch_types=scratch)


def _make_sc_deg(nchunk: int):
    mesh = plsc.VectorSubcoreMesh(
        core_axis_name="c", subcore_axis_name="s", num_cores=NC, num_subcores=NS
    )
    out_type = [jax.ShapeDtypeStruct((NC, N_PAD, DEG_W), jnp.float32)]
    scratch = [
        pltpu.VMEM((nchunk, CHUNK), jnp.int32),
        pltpu.VMEM((CHUNK, DEG_W), jnp.float32),
        pltpu.VMEM((ZROWS, DEG_W), jnp.float32),
        pltpu.VMEM_SHARED((N_PAD, DEG_W), jnp.float32),
    ]

    def body(dst_hbm, deg_hbm, didx, obuf, dzbuf, dacc):
        cid = lax.axis_index("c")
        sid = lax.axis_index("s")
        wid = sid * NC + cid
        pltpu.sync_copy(dst_hbm.at[wid], didx)

        zvec = jnp.zeros((16,), jnp.float32)
        onev = jnp.ones((16,), jnp.float32)

        def fill(j, carry):
            for k in range(DEG_W // 16):
                dzbuf[j, pl.ds(k * 16, 16)] = zvec
            for r in range(CHUNK // ZROWS):
                for k in range(DEG_W // 16):
                    obuf[j + r * ZROWS, pl.ds(k * 16, 16)] = onev
            return carry

        lax.fori_loop(0, ZROWS, fill, 0)

        base = sid * ROWS_PER_TILE
        for z in range(ROWS_PER_TILE // ZROWS):
            pltpu.sync_copy(dzbuf, dacc.at[pl.ds(base + z * ZROWS, ZROWS)])
        plsc.subcore_barrier()

        def chunk_body(j, carry):
            pltpu.sync_copy(obuf, dacc.at[didx.at[j]], add=True)
            return carry

        lax.fori_loop(0, nchunk, chunk_body, 0)
        plsc.subcore_barrier()

        pltpu.sync_copy(
            dacc.at[pl.ds(base, ROWS_PER_TILE)],
            deg_hbm.at[cid, pl.ds(base, ROWS_PER_TILE)],
        )

    return pl.kernel(body, out_type=out_type, mesh=mesh, scratch_types=scratch)


_BR = 2048


def _tc_dense(h, wl, wr, b):
    n, din = h.shape
    dout = wl.shape[1]

    def body(h_ref, wl_ref, wr_ref, b_ref, y_ref, z_ref):
        hb = h_ref[...]
        y_ref[...] = jnp.dot(hb, wl_ref[...], preferred_element_type=jnp.float32)
        z_ref[...] = (
            jnp.dot(hb, wr_ref[...], preferred_element_type=jnp.float32)
            + b_ref[...]
        )

    return pl.pallas_call(
        body,
        grid=(n // _BR,),
        in_specs=[
            pl.BlockSpec((_BR, din), lambda i: (i, 0)),
            pl.BlockSpec((din, dout), lambda i: (0, 0)),
            pl.BlockSpec((din, dout), lambda i: (0, 0)),
            pl.BlockSpec((1, dout), lambda i: (0, 0)),
        ],
        out_specs=[
            pl.BlockSpec((_BR, dout), lambda i: (i, 0)),
            pl.BlockSpec((_BR, dout), lambda i: (i, 0)),
        ],
        out_shape=[
            jax.ShapeDtypeStruct((n, dout), jnp.float32),
            jax.ShapeDtypeStruct((n, dout), jnp.float32),
        ],
    )(h, wl, wr, b.reshape(1, -1))


def _tc_combine_dense(acc, degp, z_prev, wl, wr, b):
    _, n, d = acc.shape
    dout = wl.shape[1]

    def body(a_ref, g_ref, z_ref, wl_ref, wr_ref, b_ref, y_ref, z2_ref):
        deg = jnp.maximum(g_ref[0, :, 0] + g_ref[1, :, 0], 1.0)
        hb = (a_ref[0] + a_ref[1]) / deg[:, None] + z_ref[...]
        hb = jnp.maximum(hb, 0.0)
        y_ref[...] = jnp.dot(hb, wl_ref[...], preferred_element_type=jnp.float32)
        z2_ref[...] = (
            jnp.dot(hb, wr_ref[...], preferred_element_type=jnp.float32)
            + b_ref[...]
        )

    return pl.pallas_call(
        body,
        grid=(n // _BR,),
        in_specs=[
            pl.BlockSpec((2, _BR, d), lambda i: (0, i, 0)),
            pl.BlockSpec((2, _BR, DEG_W), lambda i: (0, i, 0)),
            pl.BlockSpec((_BR, d), lambda i: (i, 0)),
            pl.BlockSpec((d, dout), lambda i: (0, 0)),
            pl.BlockSpec((d, dout), lambda i: (0, 0)),
            pl.BlockSpec((1, dout), lambda i: (0, 0)),
        ],
        out_specs=[
            pl.BlockSpec((_BR, dout), lambda i: (i, 0)),
            pl.BlockSpec((_BR, dout), lambda i: (i, 0)),
        ],
        out_shape=[
            jax.ShapeDtypeStruct((n, dout), jnp.float32),
            jax.ShapeDtypeStruct((n, dout), jnp.float32),
        ],
    )(acc, degp, z_prev, wl, wr, b.reshape(1, -1))


def _tc_combine_keep(acc, degp, z_prev, wr, b):
    _, n, d = acc.shape
    dout = wr.shape[1]

    def body(a_ref, g_ref, z_ref, wr_ref, b_ref, h_ref, z2_ref):
        deg = jnp.maximum(g_ref[0, :, 0] + g_ref[1, :, 0], 1.0)
        hb = (a_ref[0] + a_ref[1]) / deg[:, None] + z_ref[...]
        hb = jnp.maximum(hb, 0.0)
        h_ref[...] = hb
        z2_ref[...] = (
            jnp.dot(hb, wr_ref[...], preferred_element_type=jnp.float32)
            + b_ref[...]
        )

    return pl.pallas_call(
        body,
        grid=(n // _BR,),
        in_specs=[
            pl.BlockSpec((2, _BR, d), lambda i: (0, i, 0)),
            pl.BlockSpec((2, _BR, DEG_W), lambda i: (0, i, 0)),
            pl.BlockSpec((_BR, d), lambda i: (i, 0)),
            pl.BlockSpec((d, dout), lambda i: (0, 0)),
            pl.BlockSpec((1, dout), lambda i: (0, 0)),
        ],
        out_specs=[
            pl.BlockSpec((_BR, d), lambda i: (i, 0)),
            pl.BlockSpec((_BR, dout), lambda i: (i, 0)),
        ],
        out_shape=[
            jax.ShapeDtypeStruct((n, d), jnp.float32),
            jax.ShapeDtypeStruct((n, dout), jnp.float32),
        ],
    )(acc, degp, z_prev, wr, b.reshape(1, -1))


def _tc_combine_final(acc, degp, z_prev, wl):
    _, n, d = acc.shape
    dout = wl.shape[1]

    def body(a_ref, g_ref, z_ref, wl_ref, o_ref):
        deg = jnp.maximum(g_ref[0, :, 0] + g_ref[1, :, 0], 1.0)
        agg = (a_ref[0] + a_ref[1]) / deg[:, None]
        o_ref[...] = (
            jnp.dot(agg, wl_ref[...], preferred_element_type=jnp.float32)
            + z_ref[...]
        )

    return pl.pallas_call(
        body,
        grid=(n // _BR,),
        in_specs=[
            pl.BlockSpec((2, _BR, d), lambda i: (0, i, 0)),
            pl.BlockSpec((2, _BR, DEG_W), lambda i: (0, i, 0)),
            pl.BlockSpec((_BR, dout), lambda i: (i, 0)),
            pl.BlockSpec((d, dout), lambda i: (0, 0)),
        ],
        out_specs=pl.BlockSpec((_BR, dout), lambda i: (i, 0)),
        out_shape=jax.ShapeDtypeStruct((n, dout), jnp.float32),
    )(acc, degp, z_prev, wl)


def kernel(x, edge_index, W1l, b1, W1r, W2l, b2, W2r, W3l, b3, W3r):
    n = x.shape[0]
    e = edge_index.shape[1]
    per_tile = e // NW
    assert per_tile * NW == e
    pt_pad = -(-per_tile // (G * CHUNK)) * (G * CHUNK)
    nchunk = pt_pad // CHUNK

    src2 = edge_index[0].astype(jnp.int32).reshape(NW, per_tile)
    dst2 = edge_index[1].astype(jnp.int32).reshape(NW, per_tile)
    pad = ((0, 0), (0, pt_pad - per_tile))
    src3 = jnp.pad(src2, pad, constant_values=N_PAD - 1).reshape(NW, nchunk, CHUNK)
    dst3 = jnp.pad(dst2, pad, constant_values=N_PAD - 1).reshape(NW, nchunk, CHUNK)
    xp = jnp.pad(x, ((0, N_PAD - n), (0, 0)))

    d_hid = W1l.shape[1]
    agg_hid = _make_sc_agg(nchunk, d_hid)
    deg_fn = _make_sc_deg(nchunk)

    (degp,) = deg_fn(dst3)
    y1, z1 = _tc_dense(xp, W1l, W1r, b1)
    (acc1,) = agg_hid(y1, src3, dst3)
    y2, z2 = _tc_combine_dense(acc1, degp, z1, W2l, W2r, b2)
    (acc2,) = agg_hid(y2, src3, dst3)
    h2, z3 = _tc_combine_keep(acc2, degp, z2, W3r, b3)
    (acc3,) = agg_hid(h2, src3, dst3)
    return _tc_combine_final(acc3, degp, z3, W3l)[:n]


# speedup vs baseline: 1.3256x; 1.3256x over previous
import jax
import jax.numpy as jnp
from jax import lax
from jax.experimental import pallas as pl
from jax.experimental.pallas import tpu as pltpu
from jax.experimental.pallas import tpu_sc as plsc

N_PAD = 10240
NC = 2
NS = 16
NW = NC * NS
CHUNK = 128
ROWS_PER_TILE = N_PAD // NS
ZROWS = 32
DEG_W = 128


def _make_sc_agg(nchunk: int, d: int):
    mesh = plsc.VectorSubcoreMesh(
        core_axis_name="c", subcore_axis_name="s", num_cores=NC, num_subcores=NS
    )
    out_type = [jax.ShapeDtypeStruct((NC, N_PAD, d), jnp.float32)]
    scratch = [
        pltpu.VMEM((nchunk, CHUNK), jnp.int32),
        pltpu.VMEM((nchunk, CHUNK), jnp.int32),
        pltpu.VMEM((CHUNK, d), jnp.float32),
        pltpu.VMEM((ZROWS, d), jnp.float32),
        pltpu.VMEM_SHARED((N_PAD, d), jnp.float32),
        pltpu.SemaphoreType.DMA,
    ]

    def body(y_hbm, src_hbm, dst_hbm, out_hbm, sidx, didx, rows, zbuf, acc, sem):
        cid = lax.axis_index("c")
        sid = lax.axis_index("s")
        wid = sid * NC + cid

        pltpu.sync_copy(src_hbm.at[wid], sidx)
        pltpu.sync_copy(dst_hbm.at[wid], didx)

        zvec = jnp.zeros((16,), jnp.float32)

        def zfill(j, carry):
            for k in range(d // 16):
                zbuf[j, pl.ds(k * 16, 16)] = zvec
            return carry

        lax.fori_loop(0, ZROWS, zfill, 0)

        base = sid * ROWS_PER_TILE
        for z in range(ROWS_PER_TILE // ZROWS):
            pltpu.sync_copy(zbuf, acc.at[pl.ds(base + z * ZROWS, ZROWS)])
        plsc.subcore_barrier()

        def chunk_body(j, carry):
            pltpu.async_copy(y_hbm.at[sidx.at[j]], rows, sem).wait()
            pltpu.sync_copy(rows, acc.at[didx.at[j]], add=True)
            return carry

        lax.fori_loop(0, nchunk, chunk_body, 0)
        plsc.subcore_barrier()

        pltpu.sync_copy(
            acc.at[pl.ds(base, ROWS_PER_TILE)],
            out_hbm.at[cid, pl.ds(base, ROWS_PER_TILE)],
        )

    return pl.kernel(body, out_type=out_type, mesh=mesh, scratch_types=scratch)


def _make_sc_deg(nchunk: int):
    mesh = plsc.VectorSubcoreMesh(
        core_axis_name="c", subcore_axis_name="s", num_cores=NC, num_subcores=NS
    )
    out_type = [jax.ShapeDtypeStruct((NC, N_PAD, DEG_W), jnp.float32)]
    scratch = [
        pltpu.VMEM((nchunk, CHUNK), jnp.int32),
        pltpu.VMEM((CHUNK, DEG_W), jnp.float32),
        pltpu.VMEM((ZROWS, DEG_W), jnp.float32),
        pltpu.VMEM_SHARED((N_PAD, DEG_W), jnp.float32),
    ]

    def body(dst_hbm, deg_hbm, didx, obuf, dzbuf, dacc):
        cid = lax.axis_index("c")
        sid = lax.axis_index("s")
        wid = sid * NC + cid
        pltpu.sync_copy(dst_hbm.at[wid], didx)

        zvec = jnp.zeros((16,), jnp.float32)
        onev = jnp.ones((16,), jnp.float32)

        def fill(j, carry):
            for k in range(DEG_W // 16):
                dzbuf[j, pl.ds(k * 16, 16)] = zvec
            for r in range(CHUNK // ZROWS):
                for k in range(DEG_W // 16):
                    obuf[j + r * ZROWS, pl.ds(k * 16, 16)] = onev
            return carry

        lax.fori_loop(0, ZROWS, fill, 0)

        base = sid * ROWS_PER_TILE
        for z in range(ROWS_PER_TILE // ZROWS):
            pltpu.sync_copy(dzbuf, dacc.at[pl.ds(base + z * ZROWS, ZROWS)])
        plsc.subcore_barrier()

        def chunk_body(j, carry):
            pltpu.sync_copy(obuf, dacc.at[didx.at[j]], add=True)
            return carry

        lax.fori_loop(0, nchunk, chunk_body, 0)
        plsc.subcore_barrier()

        pltpu.sync_copy(
            dacc.at[pl.ds(base, ROWS_PER_TILE)],
            deg_hbm.at[cid, pl.ds(base, ROWS_PER_TILE)],
        )

    return pl.kernel(body, out_type=out_type, mesh=mesh, scratch_types=scratch)


_BR = 2048


def _tc_dense(h, wl, wr, b):
    n, din = h.shape
    dout = wl.shape[1]

    def body(h_ref, wl_ref, wr_ref, b_ref, y_ref, z_ref):
        hb = h_ref[...]
        y_ref[...] = jnp.dot(hb, wl_ref[...], preferred_element_type=jnp.float32)
        z_ref[...] = (
            jnp.dot(hb, wr_ref[...], preferred_element_type=jnp.float32)
            + b_ref[...]
        )

    return pl.pallas_call(
        body,
        grid=(n // _BR,),
        in_specs=[
            pl.BlockSpec((_BR, din), lambda i: (i, 0)),
            pl.BlockSpec((din, dout), lambda i: (0, 0)),
            pl.BlockSpec((din, dout), lambda i: (0, 0)),
            pl.BlockSpec((1, dout), lambda i: (0, 0)),
        ],
        out_specs=[
            pl.BlockSpec((_BR, dout), lambda i: (i, 0)),
            pl.BlockSpec((_BR, dout), lambda i: (i, 0)),
        ],
        out_shape=[
            jax.ShapeDtypeStruct((n, dout), jnp.float32),
            jax.ShapeDtypeStruct((n, dout), jnp.float32),
        ],
    )(h, wl, wr, b.reshape(1, -1))


def _tc_combine_dense(acc, degp, z_prev, wl, wr, b):
    _, n, d = acc.shape
    dout = wl.shape[1]

    def body(a_ref, g_ref, z_ref, wl_ref, wr_ref, b_ref, y_ref, z2_ref):
        deg = jnp.maximum(g_ref[0, :, 0] + g_ref[1, :, 0], 1.0)
        hb = (a_ref[0] + a_ref[1]) / deg[:, None] + z_ref[...]
        hb = jnp.maximum(hb, 0.0)
        y_ref[...] = jnp.dot(hb, wl_ref[...], preferred_element_type=jnp.float32)
        z2_ref[...] = (
            jnp.dot(hb, wr_ref[...], preferred_element_type=jnp.float32)
            + b_ref[...]
        )

    return pl.pallas_call(
        body,
        grid=(n // _BR,),
        in_specs=[
            pl.BlockSpec((2, _BR, d), lambda i: (0, i, 0)),
            pl.BlockSpec((2, _BR, DEG_W), lambda i: (0, i, 0)),
            pl.BlockSpec((_BR, d), lambda i: (i, 0)),
            pl.BlockSpec((d, dout), lambda i: (0, 0)),
            pl.BlockSpec((d, dout), lambda i: (0, 0)),
            pl.BlockSpec((1, dout), lambda i: (0, 0)),
        ],
        out_specs=[
            pl.BlockSpec((_BR, dout), lambda i: (i, 0)),
            pl.BlockSpec((_BR, dout), lambda i: (i, 0)),
        ],
        out_shape=[
            jax.ShapeDtypeStruct((n, dout), jnp.float32),
            jax.ShapeDtypeStruct((n, dout), jnp.float32),
        ],
    )(acc, degp, z_prev, wl, wr, b.reshape(1, -1))


def _tc_combine_keep(acc, degp, z_prev, wr, b):
    _, n, d = acc.shape
    dout = wr.shape[1]

    def body(a_ref, g_ref, z_ref, wr_ref, b_ref, h_ref, z2_ref):
        deg = jnp.maximum(g_ref[0, :, 0] + g_ref[1, :, 0], 1.0)
        hb = (a_ref[0] + a_ref[1]) / deg[:, None] + z_ref[...]
        hb = jnp.maximum(hb, 0.0)
        h_ref[...] = hb
        z2_ref[...] = (
            jnp.dot(hb, wr_ref[...], preferred_element_type=jnp.float32)
            + b_ref[...]
        )

    return pl.pallas_call(
        body,
        grid=(n // _BR,),
        in_specs=[
            pl.BlockSpec((2, _BR, d), lambda i: (0, i, 0)),
            pl.BlockSpec((2, _BR, DEG_W), lambda i: (0, i, 0)),
            pl.BlockSpec((_BR, d), lambda i: (i, 0)),
            pl.BlockSpec((d, dout), lambda i: (0, 0)),
            pl.BlockSpec((1, dout), lambda i: (0, 0)),
        ],
        out_specs=[
            pl.BlockSpec((_BR, d), lambda i: (i, 0)),
            pl.BlockSpec((_BR, dout), lambda i: (i, 0)),
        ],
        out_shape=[
            jax.ShapeDtypeStruct((n, d), jnp.float32),
            jax.ShapeDtypeStruct((n, dout), jnp.float32),
        ],
    )(acc, degp, z_prev, wr, b.reshape(1, -1))


def _tc_combine_final(acc, degp, z_prev, wl):
    _, n, d = acc.shape
    dout = wl.shape[1]

    def body(a_ref, g_ref, z_ref, wl_ref, o_ref):
        deg = jnp.maximum(g_ref[0, :, 0] + g_ref[1, :, 0], 1.0)
        agg = (a_ref[0] + a_ref[1]) / deg[:, None]
        o_ref[...] = (
            jnp.dot(agg, wl_ref[...], preferred_element_type=jnp.float32)
            + z_ref[...]
        )

    return pl.pallas_call(
        body,
        grid=(n // _BR,),
        in_specs=[
            pl.BlockSpec((2, _BR, d), lambda i: (0, i, 0)),
            pl.BlockSpec((2, _BR, DEG_W), lambda i: (0, i, 0)),
            pl.BlockSpec((_BR, dout), lambda i: (i, 0)),
            pl.BlockSpec((d, dout), lambda i: (0, 0)),
        ],
        out_specs=pl.BlockSpec((_BR, dout), lambda i: (i, 0)),
        out_shape=jax.ShapeDtypeStruct((n, dout), jnp.float32),
    )(acc, degp, z_prev, wl)


def kernel(x, edge_index, W1l, b1, W1r, W2l, b2, W2r, W3l, b3, W3r):
    n = x.shape[0]
    e = edge_index.shape[1]
    per_tile = e // NW
    assert per_tile * NW == e
    pt_pad = -(-per_tile // CHUNK) * CHUNK
    nchunk = pt_pad // CHUNK

    src2 = edge_index[0].astype(jnp.int32).reshape(NW, per_tile)
    dst2 = edge_index[1].astype(jnp.int32).reshape(NW, per_tile)
    pad = ((0, 0), (0, pt_pad - per_tile))
    src3 = jnp.pad(src2, pad, constant_values=N_PAD - 1).reshape(NW, nchunk, CHUNK)
    dst3 = jnp.pad(dst2, pad, constant_values=N_PAD - 1).reshape(NW, nchunk, CHUNK)
    xp = jnp.pad(x, ((0, N_PAD - n), (0, 0)))

    d_hid = W1l.shape[1]
    agg_hid = _make_sc_agg(nchunk, d_hid)
    deg_fn = _make_sc_deg(nchunk)

    (degp,) = deg_fn(dst3)
    y1, z1 = _tc_dense(xp, W1l, W1r, b1)
    (acc1,) = agg_hid(y1, src3, dst3)
    y2, z2 = _tc_combine_dense(acc1, degp, z1, W2l, W2r, b2)
    (acc2,) = agg_hid(y2, src3, dst3)
    h2, z3 = _tc_combine_keep(acc2, degp, z2, W3r, b3)
    (acc3,) = agg_hid(h2, src3, dst3)
    return _tc_combine_final(acc3, degp, z3, W3l)[:n]
